# baseline (speedup 1.0000x reference)
import functools

import jax
import jax.numpy as jnp
from jax import lax
from jax.experimental import pallas as pl
from jax.experimental.pallas import tpu as pltpu
from jax.experimental.pallas import tpu_sc as plsc

EPS = 1e-12
NC, NS, L = 2, 16, 16
NW = NC * NS
D = 1024
NLANE = D // L
CHUNK = 32


def _rsqrt16(v):
    i = plsc.bitcast(v, jnp.int32)
    i = jnp.full((L,), 0x5F3759DF, jnp.int32) - lax.shift_right_logical(
        i, jnp.full((L,), 1, jnp.int32)
    )
    y = plsc.bitcast(i, jnp.float32)
    half = 0.5 * v
    for _ in range(3):
        y = y * (1.5 - half * y * y)
    return y


def _sc_body(x_hbm, pe_hbm, g_hbm, b_hbm, out_hbm, xv, pev, gv, bv):
    wid = lax.axis_index("s") * NC + lax.axis_index("c")
    rows_per_w = x_hbm.shape[0] // NW
    base = wid * rows_per_w
    pbase = (wid % (pe_hbm.shape[0] // rows_per_w)) * rows_per_w

    pltpu.sync_copy(g_hbm, gv)
    pltpu.sync_copy(b_hbm, bv)

    def chunk_body(c, carry):
        start = base + c * CHUNK
        pstart = pbase + c * CHUNK
        pltpu.sync_copy(x_hbm.at[pl.ds(start, CHUNK)], xv)
        pltpu.sync_copy(pe_hbm.at[pl.ds(pstart, CHUNK)], pev)

        def row_body(r, carry2):
            acc1 = jnp.zeros((L,), jnp.float32)
            acc2 = jnp.zeros((L,), jnp.float32)
            for j in range(NLANE):
                v = xv[r, pl.ds(j * L, L)] + pev[r, pl.ds(j * L, L)]
                acc1 = acc1 + v
                acc2 = acc2 + v * v
            s1 = jnp.sum(acc1)
            s2 = jnp.sum(acc2)
            uv = jnp.full((L,), s1 * (1.0 / D), jnp.float32)
            e2 = jnp.full((L,), s2 * (1.0 / D), jnp.float32)
            var = e2 - uv * uv + EPS
            rv = _rsqrt16(var)
            for j in range(NLANE):
                sl = pl.ds(j * L, L)
                v = xv[r, sl] + pev[r, sl]
                xv[r, sl] = (v - uv) * rv * gv[sl] + bv[sl]
            return carry2

        lax.fori_loop(0, CHUNK, row_body, 0)
        pltpu.sync_copy(xv, out_hbm.at[pl.ds(start, CHUNK)])
        return carry

    lax.fori_loop(0, rows_per_w // CHUNK, chunk_body, 0)


@jax.jit
def _run_sc(xf, pos_embed, gamma, beta):
    R, _ = xf.shape
    mesh = plsc.VectorSubcoreMesh(
        core_axis_name="c", subcore_axis_name="s", num_cores=NC, num_subcores=NS
    )
    f = functools.partial(
        pl.kernel,
        out_type=jax.ShapeDtypeStruct((R, D), jnp.float32),
        mesh=mesh,
        scratch_types=[
            pltpu.VMEM((CHUNK, D), jnp.float32),
            pltpu.VMEM((CHUNK, D), jnp.float32),
            pltpu.VMEM((D,), jnp.float32),
            pltpu.VMEM((D,), jnp.float32),
        ],
        compiler_params=pltpu.CompilerParams(needs_layout_passes=False),
    )(_sc_body)
    return f(xf, pos_embed, gamma, beta)


def kernel(x, pos_embed, gamma, beta):
    B, S, Dd = x.shape
    xf = x.reshape(B * S, Dd)
    out = _run_sc(xf, pos_embed[:S], gamma, beta)
    return out.reshape(B, S, Dd)


# speedup vs baseline: 7.5133x; 7.5133x over previous
import functools

import jax
import jax.numpy as jnp
from jax.experimental import pallas as pl

EPS = 1e-12


def _ln_kernel(x_ref, pe_ref, g_ref, b_ref, o_ref):
    xb = x_ref[...] + pe_ref[...][None, :, :]
    u = jnp.mean(xb, axis=-1, keepdims=True)
    d = xb - u
    s = jnp.mean(d * d, axis=-1, keepdims=True)
    o_ref[...] = g_ref[...] * (d * jax.lax.rsqrt(s + EPS)) + b_ref[...]


@functools.partial(jax.jit, static_argnames=("s_blk",))
def _run(x, pos_embed, gamma, beta, s_blk=512):
    B, S, D = x.shape
    gamma2 = gamma.reshape(1, D)
    beta2 = beta.reshape(1, D)
    grid = (S // s_blk,)
    return pl.pallas_call(
        _ln_kernel,
        grid=grid,
        in_specs=[
            pl.BlockSpec((B, s_blk, D), lambda s: (0, s, 0)),
            pl.BlockSpec((s_blk, D), lambda s: (s, 0)),
            pl.BlockSpec((1, D), lambda s: (0, 0)),
            pl.BlockSpec((1, D), lambda s: (0, 0)),
        ],
        out_specs=pl.BlockSpec((B, s_blk, D), lambda s: (0, s, 0)),
        out_shape=jax.ShapeDtypeStruct((B, S, D), x.dtype),
    )(x, pos_embed, gamma2, beta2)


def kernel(x, pos_embed, gamma, beta):
    S = x.shape[1]
    return _run(x, pos_embed[:S], gamma, beta)
